# baseline (speedup 1.0000x reference)
import functools

import jax
import jax.numpy as jnp
from jax import lax
from jax.experimental import pallas as pl
from jax.experimental.pallas import tpu as pltpu
from jax.experimental.pallas import tpu_sc as plsc

_LANES = 16
_CHUNK = 128


def _mf_kernel(b_per_w, n_feat, uids_hbm, iids_hbm, utab_hbm,
               itab_hbm, w_hbm, out_hbm, uidx_v, iidx_v, urows_v, irows_v,
               w_v, out_v, sem):
    n_chunks = b_per_w // _CHUNK
    wid = lax.axis_index("s") * 2 + lax.axis_index("c")
    base = wid * b_per_w

    pltpu.sync_copy(uids_hbm.at[pl.ds(wid * n_chunks, n_chunks)], uidx_v)
    pltpu.sync_copy(iids_hbm.at[pl.ds(wid * n_chunks, n_chunks)], iidx_v)
    pltpu.sync_copy(w_hbm, w_v)

    copies = []
    for j in range(n_chunks):
        copies.append(pltpu.make_async_copy(
            utab_hbm.at[uidx_v.at[j]],
            urows_v.at[pl.ds(j * _CHUNK, _CHUNK)], sem))
        copies.append(pltpu.make_async_copy(
            itab_hbm.at[iidx_v.at[j]],
            irows_v.at[pl.ds(j * _CHUNK, _CHUNK)], sem))
    for c in copies:
        c.start()
    for c in copies:
        c.wait()

    w0 = w_v[pl.ds(0, _LANES)]
    w1 = w_v[pl.ds(_LANES, _LANES)]
    lane = jax.lax.iota(jnp.int32, _LANES)
    perms = [lane ^ k for k in (8, 4, 2, 1)]

    unroll = 16

    dnums = lax.GatherDimensionNumbers(
        offset_dims=(), collapsed_slice_dims=(0,), start_index_map=(0,))

    def permute(t, p):
        return lax.gather(t, p[:, None], dnums, slice_sizes=(1,),
                          mode=lax.GatherScatterMode.PROMISE_IN_BOUNDS)

    def rowsum(t):
        for p in perms:
            t = t + permute(t, p)
        return t

    def body(blk, carry):
        block = None
        for r in range(unroll):
            b = blk * unroll + r
            u0 = urows_v[b, pl.ds(0, _LANES)]
            u1 = urows_v[b, pl.ds(_LANES, _LANES)]
            v0 = irows_v[b, pl.ds(0, _LANES)]
            v1 = irows_v[b, pl.ds(_LANES, _LANES)]
            t = u0 * v0 * w0 + u1 * v1 * w1
            s = rowsum(t)
            block = s if block is None else jnp.where(lane == r, s, block)
        out_v[pl.ds(blk * unroll, unroll)] = block
        return carry

    lax.fori_loop(0, b_per_w // unroll, body, 0)

    pltpu.sync_copy(out_v, out_hbm.at[pl.ds(base, b_per_w)])


def kernel(user_ids, item_ids, user_table, item_table, predict_w):
    batch = user_ids.shape[0]
    n_feat = user_table.shape[1]
    info = plsc.get_sparse_core_info()
    n_workers = info.num_cores * info.num_subcores
    b_per_w = batch // n_workers
    n_chunks = b_per_w // _CHUNK

    uids = user_ids.astype(jnp.int32).reshape(n_workers * n_chunks, _CHUNK)
    iids = item_ids.astype(jnp.int32).reshape(n_workers * n_chunks, _CHUNK)
    w = predict_w.reshape(n_feat).astype(jnp.float32)

    mesh = plsc.VectorSubcoreMesh(core_axis_name="c", subcore_axis_name="s")
    run = pl.kernel(
        functools.partial(_mf_kernel, b_per_w, n_feat),
        mesh=mesh,
        compiler_params=pltpu.CompilerParams(use_tc_tiling_on_sc=False),
        out_type=jax.ShapeDtypeStruct((batch,), jnp.float32),
        scratch_types=[
            pltpu.VMEM((n_chunks, _CHUNK), jnp.int32),
            pltpu.VMEM((n_chunks, _CHUNK), jnp.int32),
            pltpu.VMEM((b_per_w, n_feat), jnp.float32),
            pltpu.VMEM((b_per_w, n_feat), jnp.float32),
            pltpu.VMEM((n_feat,), jnp.float32),
            pltpu.VMEM((b_per_w,), jnp.float32),
            pltpu.SemaphoreType.DMA,
        ],
    )
    return run(uids, iids, user_table, item_table, w)
